# Initial kernel scaffold; baseline (speedup 1.0000x reference)
import jax
import jax.numpy as jnp
from jax.experimental import pallas as pl


def kernel(x, pos, table):
    raise NotImplementedError("write your pallas kernel here")


# speedup vs baseline: 7.1616x; 7.1616x over previous
import jax
import jax.numpy as jnp
from jax.experimental import pallas as pl

_POS_RFACTOR = 8
_RPAD = 32
_REP = 64
_HBLK = 2048


def _embed_add_kernel(idx_ref, tabT_ref, x_ref, o_ref):
    j = pl.program_id(1)
    hp = idx_ref.shape[-1]
    g = _HBLK // _REP
    idx_row = idx_ref[0] // _POS_RFACTOR
    iota_r = jax.lax.broadcasted_iota(jnp.int32, (_RPAD, hp), 0)
    oh = (iota_r == idx_row).astype(jnp.float32)
    e = jnp.dot(tabT_ref[...], oh, preferred_element_type=jnp.float32)
    iota_p = jax.lax.broadcasted_iota(jnp.int32, (hp, _HBLK), 0)
    iota_h = jax.lax.broadcasted_iota(jnp.int32, (hp, _HBLK), 1)
    sel = (iota_p == j * g + iota_h // _REP).astype(jnp.float32)
    emb = jnp.dot(e, sel, preferred_element_type=jnp.float32)
    o_ref[0] = x_ref[0] + emb


def kernel(x, pos, table):
    b, d, h = x.shape
    hp = pos.shape[2]
    rows = table.shape[0]
    idx = pos[0, :, :, 0].astype(jnp.int32).reshape(b, 1, hp)
    tab_t = jnp.zeros((d, _RPAD), jnp.float32).at[:, :rows].set(table.T)
    return pl.pallas_call(
        _embed_add_kernel,
        grid=(b, h // _HBLK),
        in_specs=[
            pl.BlockSpec((1, 1, hp), lambda bi, ji: (bi, 0, 0)),
            pl.BlockSpec((d, _RPAD), lambda bi, ji: (0, 0)),
            pl.BlockSpec((1, d, _HBLK), lambda bi, ji: (bi, 0, ji)),
        ],
        out_specs=pl.BlockSpec((1, d, _HBLK), lambda bi, ji: (bi, 0, ji)),
        out_shape=jax.ShapeDtypeStruct(x.shape, x.dtype),
    )(idx, tab_t, x)
